# baseline (speedup 1.0000x reference)
import functools

import jax
import jax.numpy as jnp
from jax import lax
from jax.experimental import pallas as pl
from jax.experimental.pallas import tpu as pltpu
from jax.experimental.pallas import tpu_sc as plsc

_NC = 2
_NS = 16
_NW = _NC * _NS
_C = 128
_L = 16


@functools.partial(jax.jit, static_argnums=(2,))
def _gather_t(cap_t, table, nseq):
    d = table.shape[1]
    b = cap_t.shape[1]
    mesh = plsc.VectorSubcoreMesh(core_axis_name="c", subcore_axis_name="s")

    @functools.partial(
        pl.kernel,
        out_type=jax.ShapeDtypeStruct((nseq, d, b), jnp.float32),
        mesh=mesh,
        scratch_types=[
            pltpu.VMEM((nseq, _C), jnp.int32),
            pltpu.VMEM((2, d, _C), jnp.float32),
            pltpu.SemaphoreType.DMA,
            pltpu.SemaphoreType.DMA,
        ],
        compiler_params=pltpu.CompilerParams(needs_layout_passes=False),
    )
    def k(cap_hbm, tab_hbm, out_hbm, idx_v, tile_v, g0, g1):
        wid = lax.axis_index("s") * _NC + lax.axis_index("c")
        b0 = wid * _C
        gsems = (g0, g1)
        pltpu.sync_copy(cap_hbm.at[:, pl.ds(b0, _C)], idx_v)

        def fire(s, buf):
            @pl.loop(0, _C, step=_L)
            def _(i0):
                vec = idx_v[s, pl.ds(i0, _L)]
                for i in range(_L):
                    pltpu.async_copy(
                        tab_hbm.at[vec[i]],
                        tile_v.at[buf, :, i0 + i],
                        gsems[buf],
                    )

        def wait_gather(buf):
            pltpu.make_async_copy(
                tab_hbm.at[pl.ds(0, _C)], tile_v.at[buf], gsems[buf]
            ).wait()

        def store(s, buf):
            pltpu.sync_copy(tile_v.at[buf], out_hbm.at[s, :, pl.ds(b0, _C)])

        fire(0, 0)
        fire(1, 1)

        @pl.loop(0, nseq - 2, step=2)
        def _(jj):
            for buf in range(2):
                s = jj + buf
                wait_gather(buf)
                store(s, buf)
                fire(s + 2, buf)

        for buf in range(2):
            s = nseq - 2 + buf
            wait_gather(buf)
            store(s, buf)

    return k(cap_t, table)


def kernel(captions, weights):
    bsz, seq = captions.shape
    cap_t = captions.T.astype(jnp.int32)
    out_t = _gather_t(cap_t, weights, seq)
    return out_t.transpose(2, 0, 1)


# speedup vs baseline: 1.9354x; 1.9354x over previous
import functools

import jax
import jax.numpy as jnp
from jax import lax
from jax.experimental import pallas as pl
from jax.experimental.pallas import tpu as pltpu
from jax.experimental.pallas import tpu_sc as plsc

_NC = 2
_NS = 16
_NW = _NC * _NS
_C = 128
_L = 16


@functools.partial(jax.jit, static_argnums=(2,))
def _gather_t(cap_t, wtp, nseq):
    d2 = wtp.shape[1]
    d = d2 // 2
    b = cap_t.shape[1]
    mesh = plsc.VectorSubcoreMesh(core_axis_name="c", subcore_axis_name="s")

    @functools.partial(
        pl.kernel,
        out_type=jax.ShapeDtypeStruct((nseq, d, b), jnp.float32),
        mesh=mesh,
        scratch_types=[
            pltpu.VMEM((nseq, _C), jnp.int32),
            pltpu.VMEM((nseq, _C), jnp.int32),
            pltpu.VMEM((2, _C, d2), jnp.float32),
            pltpu.VMEM((2, d, _C), jnp.float32),
            pltpu.SemaphoreType.DMA,
            pltpu.SemaphoreType.DMA,
        ],
        compiler_params=pltpu.CompilerParams(needs_layout_passes=False),
    )
    def k(cap_hbm, wtp_hbm, out_hbm, idx2_v, col_v, rows_v, tile_v, g0, g1):
        wid = lax.axis_index("s") * _NC + lax.axis_index("c")
        b0 = wid * _C
        gsems = (g0, g1)
        pltpu.sync_copy(cap_hbm.at[:, pl.ds(b0, _C)], idx2_v)

        @pl.loop(0, nseq)
        def _(s):
            for i0 in range(0, _C, _L):
                v = idx2_v[s, pl.ds(i0, _L)]
                col_v[s, pl.ds(i0, _L)] = (v & 1) * d
                idx2_v[s, pl.ds(i0, _L)] = lax.shift_right_logical(v, 1)

        rowvs = [lax.iota(jnp.int32, _L) + i0 for i0 in range(0, _C, _L)]

        def fire(s, buf):
            pltpu.async_copy(
                wtp_hbm.at[idx2_v.at[s]], rows_v.at[buf], gsems[buf]
            )

        def wait_gather(buf):
            pltpu.make_async_copy(
                wtp_hbm.at[pl.ds(0, _C)], rows_v.at[buf], gsems[buf]
            ).wait()

        def transpose(s, buf):
            src = rows_v.at[buf]
            for q, rowv in enumerate(rowvs):
                hv = col_v[s, pl.ds(q * _L, _L)]

                @pl.loop(0, d, step=8)
                def _(d0):
                    for dj in range(8):
                        v = plsc.load_gather(src, [rowv, hv + (d0 + dj)])
                        tile_v[buf, d0 + dj, pl.ds(q * _L, _L)] = v

        def store(s, buf):
            pltpu.sync_copy(tile_v.at[buf], out_hbm.at[s, :, pl.ds(b0, _C)])

        fire(0, 0)
        fire(1, 1)

        @pl.loop(0, nseq - 2, step=2)
        def _(jj):
            for buf in range(2):
                s = jj + buf
                wait_gather(buf)
                transpose(s, buf)
                fire(s + 2, buf)
                store(s, buf)

        for buf in range(2):
            s = nseq - 2 + buf
            wait_gather(buf)
            transpose(s, buf)
            store(s, buf)

    return k(cap_t, wtp)


def kernel(captions, weights):
    bsz, seq = captions.shape
    v, d = weights.shape
    cap_t = captions.T.astype(jnp.int32)
    wtp = weights.reshape(v // 2, 2 * d)
    out_t = _gather_t(cap_t, wtp, seq)
    return out_t.transpose(2, 0, 1)


# speedup vs baseline: 3.8986x; 2.0144x over previous
import functools

import jax
import jax.numpy as jnp
from jax import lax
from jax.experimental import pallas as pl
from jax.experimental.pallas import tpu as pltpu
from jax.experimental.pallas import tpu_sc as plsc

_NC = 2
_NS = 16
_NW = _NC * _NS
_C = 128
_L = 16


@functools.partial(jax.jit, static_argnums=(2,))
def _gather_sbd(cap_t, table, nseq):
    d = table.shape[1]
    b = cap_t.shape[1]
    mesh = plsc.VectorSubcoreMesh(core_axis_name="c", subcore_axis_name="s")

    @functools.partial(
        pl.kernel,
        out_type=jax.ShapeDtypeStruct((nseq, b, d), jnp.float32),
        mesh=mesh,
        scratch_types=[
            pltpu.VMEM((nseq, _C), jnp.int32),
            pltpu.VMEM((2, _C, d), jnp.float32),
            pltpu.SemaphoreType.DMA,
            pltpu.SemaphoreType.DMA,
        ],
        compiler_params=pltpu.CompilerParams(needs_layout_passes=False),
    )
    def k(cap_hbm, tab_hbm, out_hbm, idx_v, rows_v, g0, g1):
        wid = lax.axis_index("s") * _NC + lax.axis_index("c")
        b0 = wid * _C
        gsems = (g0, g1)
        pltpu.sync_copy(cap_hbm.at[:, pl.ds(b0, _C)], idx_v)

        def fire(s, buf):
            @pl.loop(0, _C, step=_L)
            def _(i0):
                vec = idx_v[s, pl.ds(i0, _L)]
                for i in range(_L):
                    pltpu.async_copy(
                        tab_hbm.at[vec[i]],
                        rows_v.at[buf, i0 + i],
                        gsems[buf],
                    )

        def wait_gather(buf):
            pltpu.make_async_copy(
                tab_hbm.at[pl.ds(0, _C)], rows_v.at[buf], gsems[buf]
            ).wait()

        def store(s, buf):
            pltpu.sync_copy(rows_v.at[buf], out_hbm.at[s, pl.ds(b0, _C)])

        fire(0, 0)
        fire(1, 1)

        @pl.loop(0, nseq - 2, step=2)
        def _(jj):
            for buf in range(2):
                s = jj + buf
                wait_gather(buf)
                store(s, buf)
                fire(s + 2, buf)

        for buf in range(2):
            s = nseq - 2 + buf
            wait_gather(buf)
            store(s, buf)

    return k(cap_t, table)


def kernel(captions, weights):
    bsz, seq = captions.shape
    cap_t = captions.T.astype(jnp.int32)
    out_sbd = _gather_sbd(cap_t, weights, seq)
    return out_sbd.transpose(1, 0, 2)


# speedup vs baseline: 5.2641x; 1.3503x over previous
import functools

import jax
import jax.numpy as jnp
from jax import lax
from jax.experimental import pallas as pl
from jax.experimental.pallas import tpu as pltpu
from jax.experimental.pallas import tpu_sc as plsc

_NC = 2
_NS = 16
_NW = _NC * _NS
_C = 128
_L = 16


@functools.partial(jax.jit, static_argnums=(2,))
def _gather_sbd(cap_t, table, nseq):
    d = table.shape[1]
    b = cap_t.shape[1]
    mesh = plsc.VectorSubcoreMesh(core_axis_name="c", subcore_axis_name="s")

    @functools.partial(
        pl.kernel,
        out_type=jax.ShapeDtypeStruct((nseq, b, d), jnp.float32),
        mesh=mesh,
        scratch_types=[
            pltpu.VMEM((nseq, _C), jnp.int32),
            pltpu.VMEM((2, _C, d), jnp.float32),
            pltpu.SemaphoreType.DMA,
            pltpu.SemaphoreType.DMA,
        ],
        compiler_params=pltpu.CompilerParams(needs_layout_passes=False),
    )
    def k(cap_hbm, tab_hbm, out_hbm, idx_v, rows_v, g0, g1):
        wid = lax.axis_index("s") * _NC + lax.axis_index("c")
        b0 = wid * _C
        gsems = (g0, g1)
        pltpu.sync_copy(cap_hbm.at[:, pl.ds(b0, _C)], idx_v)

        def fire(s, buf):
            @pl.loop(0, _C, step=_L)
            def _(i0):
                vec = idx_v[s, pl.ds(i0, _L)]
                for i in range(_L):
                    pltpu.async_copy(
                        tab_hbm.at[vec[i]],
                        rows_v.at[buf, i0 + i],
                        gsems[buf],
                    )

        def wait_gather(buf):
            pltpu.make_async_copy(
                tab_hbm.at[pl.ds(0, _C)], rows_v.at[buf], gsems[buf]
            ).wait()

        def store(s, buf):
            pltpu.sync_copy(rows_v.at[buf], out_hbm.at[s, pl.ds(b0, _C)])

        fire(0, 0)
        fire(1, 1)

        @pl.loop(0, nseq - 2, step=2)
        def _(jj):
            for buf in range(2):
                s = jj + buf
                wait_gather(buf)
                store(s, buf)
                fire(s + 2, buf)

        for buf in range(2):
            s = nseq - 2 + buf
            wait_gather(buf)
            store(s, buf)

    return k(cap_t, table)


def kernel(captions, weights):
    bsz, seq = captions.shape
    cap_t = captions.T.astype(jnp.int32)
    wt = lax.optimization_barrier(weights.T).T
    out_sbd = _gather_sbd(cap_t, wt, seq)
    return out_sbd.transpose(1, 0, 2)


# speedup vs baseline: 5.2863x; 1.0042x over previous
import functools

import jax
import jax.numpy as jnp
from jax import lax
from jax.experimental import pallas as pl
from jax.experimental.pallas import tpu as pltpu
from jax.experimental.pallas import tpu_sc as plsc

_NC = 2
_NS = 16
_NW = _NC * _NS
_C = 128
_L = 16


@functools.partial(jax.jit, static_argnums=(2,))
def _gather_sbd(cap_t, table, nseq):
    d = table.shape[1]
    b = cap_t.shape[1]
    mesh = plsc.VectorSubcoreMesh(core_axis_name="c", subcore_axis_name="s")

    @functools.partial(
        pl.kernel,
        out_type=jax.ShapeDtypeStruct((nseq, b, d), jnp.float32),
        mesh=mesh,
        scratch_types=[
            pltpu.VMEM((nseq, _C), jnp.int32),
            pltpu.VMEM((4, _C, d), jnp.float32),
            pltpu.SemaphoreType.DMA,
            pltpu.SemaphoreType.DMA,
            pltpu.SemaphoreType.DMA,
            pltpu.SemaphoreType.DMA,
            pltpu.SemaphoreType.DMA,
            pltpu.SemaphoreType.DMA,
            pltpu.SemaphoreType.DMA,
            pltpu.SemaphoreType.DMA,
        ],
        compiler_params=pltpu.CompilerParams(needs_layout_passes=False),
    )
    def k(cap_hbm, tab_hbm, out_hbm, idx_v, rows_v,
          g0, g1, g2, g3, o0, o1, o2, o3):
        wid = lax.axis_index("s") * _NC + lax.axis_index("c")
        b0 = wid * _C
        gsems = (g0, g1, g2, g3)
        osems = (o0, o1, o2, o3)
        pltpu.sync_copy(cap_hbm.at[:, pl.ds(b0, _C)], idx_v)

        def fire(s, buf):
            @pl.loop(0, _C, step=_L)
            def _(i0):
                vec = idx_v[s, pl.ds(i0, _L)]
                for i in range(_L):
                    pltpu.async_copy(
                        tab_hbm.at[vec[i]],
                        rows_v.at[buf, i0 + i],
                        gsems[buf],
                    )

        def wait_gather(buf):
            pltpu.make_async_copy(
                tab_hbm.at[pl.ds(0, _C)], rows_v.at[buf], gsems[buf]
            ).wait()

        def store(s, buf):
            pltpu.async_copy(
                rows_v.at[buf], out_hbm.at[s, pl.ds(b0, _C)], osems[buf]
            )

        def wait_store(buf):
            pltpu.make_async_copy(
                tab_hbm.at[pl.ds(0, _C)], rows_v.at[buf], osems[buf]
            ).wait()

        fire(0, 0)
        fire(1, 1)
        for s in (0, 1, 2, 3):
            wait_gather(s % 4)
            store(s, s % 4)
            if s >= 2:
                wait_store((s + 2) % 4)
            fire(s + 2, (s + 2) % 4)

        @pl.loop(4, nseq - 2, step=4)
        def _(jj):
            for u in range(4):
                s = jj + u
                buf = u
                buf2 = (u + 2) % 4
                wait_gather(buf)
                store(s, buf)
                wait_store(buf2)
                fire(s + 2, buf2)

        for s in (nseq - 2, nseq - 1):
            wait_gather(s % 4)
            store(s, s % 4)

        for buf in range(4):
            wait_store(buf)

    return k(cap_t, table)


def kernel(captions, weights):
    bsz, seq = captions.shape
    cap_t = captions.T.astype(jnp.int32)
    wt = lax.optimization_barrier(weights.T).T
    out_sbd = _gather_sbd(cap_t, wt, seq)
    return out_sbd.transpose(1, 0, 2)
